# baseline (speedup 1.0000x reference)
import jax
import jax.numpy as jnp
from jax import lax
from jax.experimental import pallas as pl
from jax.experimental.pallas import tpu as pltpu
from jax.experimental.pallas import tpu_sc as plsc

VOCAB = 99996
DIM = 64
BATCH = 4096
FIELDS = 26

NC = 2
NS = 16
NW = NC * NS
B_PER_W = BATCH // NW
ROWS_PER_CHUNK = 16
NCHUNK = B_PER_W // ROWS_PER_CHUNK
SUB = 4
ROWS_PER_SUB = ROWS_PER_CHUNK // SUB
NVREG = DIM // 16
NBUF = 4
NGRP = NCHUNK // NBUF


def _fm_body(x_hbm, table_hbm, out_hbm, idx_v, rows_v, out_v,
             sem0, sem1, sem2, sem3):
    wid = lax.axis_index("s") * NC + lax.axis_index("c")
    base_row = wid * B_PER_W
    pltpu.sync_copy(x_hbm.at[pl.ds(base_row, B_PER_W)], idx_v)
    lane = lax.broadcasted_iota(jnp.int32, (16,), 0)
    perms = [lane ^ sh for sh in (8, 4, 2, 1)]
    sems = (sem0, sem1, sem2, sem3)

    def fire(c, buf):
        for j in range(ROWS_PER_CHUNK):
            pltpu.async_copy(
                table_hbm.at[idx_v.at[c * ROWS_PER_CHUNK + j]],
                rows_v.at[buf, j],
                sems[buf],
            )

    def drain(buf):
        for j in range(ROWS_PER_CHUNK):
            pltpu.make_async_copy(
                table_hbm.at[pl.ds(0, FIELDS)],
                rows_v.at[buf, j],
                sems[buf],
            ).wait()

    def compute(c, buf):
        def jbody(j, out_vec):
            acc = [jnp.zeros((16,), jnp.float32) for _ in range(NVREG)]
            accq = jnp.zeros((16,), jnp.float32)
            for f in range(FIELDS):
                for i in range(NVREG):
                    v = rows_v[buf, j, f, pl.ds(i * 16, 16)]
                    acc[i] = acc[i] + v
                    accq = accq + v * v
            tot = -accq
            for i in range(NVREG):
                tot = tot + acc[i] * acc[i]
            for p in perms:
                tot = tot + tot.at[p].get(mode="promise_in_bounds")
            return jnp.where(lane == j, tot, out_vec)

        out_v[pl.ds(c * ROWS_PER_CHUNK, ROWS_PER_CHUNK)] = lax.fori_loop(
            0, ROWS_PER_CHUNK, jbody, jnp.zeros((16,), jnp.float32))

    for b in range(NBUF - 1):
        fire(b, b)

    def group_body(g, carry):
        for b in range(NBUF):
            c = NBUF * g + b
            drain(b)
            compute(c, b)
            nxt = c + NBUF - 1

            @pl.when(nxt < NCHUNK)
            def _():
                fire(nxt, (b + NBUF - 1) % NBUF)

        return carry

    lax.fori_loop(0, NGRP, group_body, 0)
    pltpu.sync_copy(out_v, out_hbm.at[pl.ds(base_row, B_PER_W)])


@jax.jit
def kernel(x, table):
    mesh = plsc.VectorSubcoreMesh(core_axis_name="c", subcore_axis_name="s")
    fm = pl.kernel(
        _fm_body,
        out_type=jax.ShapeDtypeStruct((BATCH,), jnp.float32),
        mesh=mesh,
        scratch_types=[
            pltpu.VMEM((B_PER_W, FIELDS), jnp.int32),
            pltpu.VMEM((NBUF, ROWS_PER_CHUNK, FIELDS, DIM), jnp.float32),
            pltpu.VMEM((B_PER_W,), jnp.float32),
            pltpu.SemaphoreType.DMA,
            pltpu.SemaphoreType.DMA,
            pltpu.SemaphoreType.DMA,
            pltpu.SemaphoreType.DMA,
        ],
        compiler_params=pltpu.CompilerParams(use_tc_tiling_on_sc=False),
    )
    return fm(x.astype(jnp.int32), table)


# speedup vs baseline: 1.0296x; 1.0296x over previous
import jax
import jax.numpy as jnp
from jax import lax
from jax.experimental import pallas as pl
from jax.experimental.pallas import tpu as pltpu
from jax.experimental.pallas import tpu_sc as plsc

VOCAB = 99996
DIM = 64
BATCH = 4096
FIELDS = 26

NC = 2
NS = 16
NW = NC * NS
B_PER_W = BATCH // NW
ROWS_PER_CHUNK = 8
NCHUNK = B_PER_W // ROWS_PER_CHUNK
NVREG = DIM // 16
NBUF = 8
NGRP = NCHUNK // NBUF


def _fm_body(x_hbm, table_hbm, out_hbm, idx_v, rows_v, out_v,
             sem0, sem1, sem2, sem3, sem4, sem5, sem6, sem7):
    wid = lax.axis_index("s") * NC + lax.axis_index("c")
    base_row = wid * B_PER_W
    pltpu.sync_copy(x_hbm.at[pl.ds(base_row, B_PER_W)], idx_v)
    lane = lax.broadcasted_iota(jnp.int32, (16,), 0)
    perms = [lane ^ sh for sh in (8, 4, 2, 1)]
    sems = (sem0, sem1, sem2, sem3, sem4, sem5, sem6, sem7)

    def fire(c, buf):
        for j in range(ROWS_PER_CHUNK):
            pltpu.async_copy(
                table_hbm.at[idx_v.at[c * ROWS_PER_CHUNK + j]],
                rows_v.at[buf, j],
                sems[buf],
            )

    def drain(buf):
        for j in range(ROWS_PER_CHUNK):
            pltpu.make_async_copy(
                table_hbm.at[pl.ds(0, FIELDS)],
                rows_v.at[buf, j],
                sems[buf],
            ).wait()

    def compute(buf, loff):
        def jbody(j, out_vec):
            acc = [jnp.zeros((16,), jnp.float32) for _ in range(NVREG)]
            accq = [jnp.zeros((16,), jnp.float32) for _ in range(NVREG)]
            for f in range(FIELDS):
                for i in range(NVREG):
                    v = rows_v[buf, j, f, pl.ds(i * 16, 16)]
                    acc[i] = acc[i] + v
                    accq[i] = accq[i] + v * v
            tot = -(accq[0] + accq[1]) - (accq[2] + accq[3])
            for i in range(NVREG):
                tot = tot + acc[i] * acc[i]
            for p in perms:
                tot = tot + tot.at[p].get(mode="promise_in_bounds")
            return jnp.where(lane == loff + j, tot, out_vec)

        return lax.fori_loop(0, ROWS_PER_CHUNK, jbody,
                             jnp.zeros((16,), jnp.float32))

    for b in range(NBUF - 1):
        fire(b, b)

    def group_body(g, carry):
        pending = None
        for b in range(NBUF):
            c = NBUF * g + b
            drain(b)
            vec = compute(b, (b % 2) * ROWS_PER_CHUNK)
            if b % 2 == 0:
                pending = vec
            else:
                out_v[pl.ds((c // 2) * 16, 16)] = pending + vec
            nxt = c + NBUF - 1

            @pl.when(nxt < NCHUNK)
            def _():
                fire(nxt, (b + NBUF - 1) % NBUF)

        return carry

    lax.fori_loop(0, NGRP, group_body, 0)
    pltpu.sync_copy(out_v, out_hbm.at[pl.ds(base_row, B_PER_W)])


@jax.jit
def kernel(x, table):
    mesh = plsc.VectorSubcoreMesh(core_axis_name="c", subcore_axis_name="s")
    fm = pl.kernel(
        _fm_body,
        out_type=jax.ShapeDtypeStruct((BATCH,), jnp.float32),
        mesh=mesh,
        scratch_types=[
            pltpu.VMEM((B_PER_W, FIELDS), jnp.int32),
            pltpu.VMEM((NBUF, ROWS_PER_CHUNK, FIELDS, DIM), jnp.float32),
            pltpu.VMEM((B_PER_W,), jnp.float32),
            pltpu.SemaphoreType.DMA,
            pltpu.SemaphoreType.DMA,
            pltpu.SemaphoreType.DMA,
            pltpu.SemaphoreType.DMA,
            pltpu.SemaphoreType.DMA,
            pltpu.SemaphoreType.DMA,
            pltpu.SemaphoreType.DMA,
            pltpu.SemaphoreType.DMA,
        ],
        compiler_params=pltpu.CompilerParams(use_tc_tiling_on_sc=False),
    )
    return fm(x.astype(jnp.int32), table)


# speedup vs baseline: 1.0501x; 1.0199x over previous
import jax
import jax.numpy as jnp
from jax import lax
from jax.experimental import pallas as pl
from jax.experimental.pallas import tpu as pltpu
from jax.experimental.pallas import tpu_sc as plsc

VOCAB = 99996
DIM = 64
BATCH = 4096
FIELDS = 26

NC = 2
NS = 16
NW = NC * NS
B_PER_W = BATCH // NW
ROWS_PER_CHUNK = 16
NCHUNK = B_PER_W // ROWS_PER_CHUNK
SUB = 4
ROWS_PER_SUB = ROWS_PER_CHUNK // SUB
NVREG = DIM // 16
NBUF = 4
NGRP = NCHUNK // NBUF


def _fm_body(x_hbm, table_hbm, out_hbm, idx_v, rows_v, out_v,
             sem0, sem1, sem2, sem3):
    wid = lax.axis_index("s") * NC + lax.axis_index("c")
    base_row = wid * B_PER_W
    pltpu.sync_copy(x_hbm.at[pl.ds(base_row, B_PER_W)], idx_v)
    lane = lax.broadcasted_iota(jnp.int32, (16,), 0)
    perms = [lane ^ sh for sh in (8, 4, 2, 1)]
    sems = (sem0, sem1, sem2, sem3)

    def fire(c, buf):
        for j in range(ROWS_PER_CHUNK):
            pltpu.async_copy(
                table_hbm.at[idx_v.at[c * ROWS_PER_CHUNK + j]],
                rows_v.at[buf, j],
                sems[buf],
            )

    def drain(buf):
        for j in range(ROWS_PER_CHUNK):
            pltpu.make_async_copy(
                table_hbm.at[pl.ds(0, FIELDS)],
                rows_v.at[buf, j],
                sems[buf],
            ).wait()

    def compute(c, buf):
        def jbody(j, out_vec):
            acc = [jnp.zeros((16,), jnp.float32) for _ in range(NVREG)]
            accq = [jnp.zeros((16,), jnp.float32) for _ in range(NVREG)]
            for f in range(FIELDS):
                for i in range(NVREG):
                    v = rows_v[buf, j, f, pl.ds(i * 16, 16)]
                    acc[i] = acc[i] + v
                    accq[i] = accq[i] + v * v
            tot = -(accq[0] + accq[1]) - (accq[2] + accq[3])
            for i in range(NVREG):
                tot = tot + acc[i] * acc[i]
            for p in perms:
                tot = tot + tot.at[p].get(mode="promise_in_bounds")
            return jnp.where(lane == j, tot, out_vec)

        out_v[pl.ds(c * ROWS_PER_CHUNK, ROWS_PER_CHUNK)] = lax.fori_loop(
            0, ROWS_PER_CHUNK, jbody, jnp.zeros((16,), jnp.float32))

    for b in range(NBUF - 1):
        fire(b, b)

    def group_body(g, carry):
        for b in range(NBUF):
            c = NBUF * g + b
            drain(b)
            compute(c, b)
            nxt = c + NBUF - 1

            @pl.when(nxt < NCHUNK)
            def _():
                fire(nxt, (b + NBUF - 1) % NBUF)

        return carry

    lax.fori_loop(0, NGRP, group_body, 0)
    pltpu.sync_copy(out_v, out_hbm.at[pl.ds(base_row, B_PER_W)])


@jax.jit
def kernel(x, table):
    mesh = plsc.VectorSubcoreMesh(core_axis_name="c", subcore_axis_name="s")
    fm = pl.kernel(
        _fm_body,
        out_type=jax.ShapeDtypeStruct((BATCH,), jnp.float32),
        mesh=mesh,
        scratch_types=[
            pltpu.VMEM((B_PER_W, FIELDS), jnp.int32),
            pltpu.VMEM((NBUF, ROWS_PER_CHUNK, FIELDS, DIM), jnp.float32),
            pltpu.VMEM((B_PER_W,), jnp.float32),
            pltpu.SemaphoreType.DMA,
            pltpu.SemaphoreType.DMA,
            pltpu.SemaphoreType.DMA,
            pltpu.SemaphoreType.DMA,
        ],
        compiler_params=pltpu.CompilerParams(use_tc_tiling_on_sc=False),
    )
    return fm(x.astype(jnp.int32), table)
